# baseline (speedup 1.0000x reference)
import jax
import jax.numpy as jnp
from jax import lax
from jax.experimental import pallas as pl
from jax.experimental.pallas import tpu as pltpu
from jax.experimental.pallas import tpu_sc as plsc

D_MODEL = 512
SEQ = 200
LANES = 16
NLG = D_MODEL // LANES
CHUNK = 40
NCHUNKS = SEQ // CHUNK
SEQS_PER_TILE = 32
EPS = 1e-5
QB = 25


def _rsqrt_nr(x):
    i = lax.bitcast_convert_type(x, jnp.int32)
    i = jnp.int32(0x5F3759DF) - (i >> 1)
    y = lax.bitcast_convert_type(i, jnp.float32)
    for _ in range(2):
        y = y * (1.5 - 0.5 * x * y * y)
    return y


def _sc_embed_ln(labels_hbm, table_hbm, pos_hbm, out_hbm,
                 idx_all, rows0, rows1, rows2, rows3, pos_v,
                 gsem0, gsem1, gsem2, gsem3, ssem0, ssem1, ssem2, ssem3):
    cid = lax.axis_index("c")
    sid = lax.axis_index("s")
    wid = sid * 2 + cid
    rows = (rows0, rows1, rows2, rows3)
    gsems = (gsem0, gsem1, gsem2, gsem3)
    ssems = (ssem0, ssem1, ssem2, ssem3)
    tile_tokens = SEQS_PER_TILE * SEQ
    pltpu.sync_copy(labels_hbm.at[pl.ds(wid * tile_tokens, tile_tokens)],
                    idx_all)

    lane = lax.iota(jnp.int32, LANES)
    perms = [(lane ^ s)[:, None] for s in (8, 4, 2, 1)]
    _gd = lax.GatherDimensionNumbers(
        offset_dims=(), collapsed_slice_dims=(0,), start_index_map=(0,))

    def hsum(v):
        for p in perms:
            v = v + lax.gather(v, p, _gd, slice_sizes=(1,),
                               mode=lax.GatherScatterMode.PROMISE_IN_BOUNDS)
        return v

    def compute(rows_v):
        @plsc.parallel_loop(0, CHUNK, unroll=2)
        def row_body(r):
            ys = []
            acc = jnp.zeros((LANES,), jnp.float32)
            acc2 = jnp.zeros((LANES,), jnp.float32)
            for i in range(NLG):
                x = rows_v[r, pl.ds(i * LANES, LANES)]
                p = pos_v[r, pl.ds(i * LANES, LANES)]
                y = x + p
                ys.append(y)
                acc = acc + y
                acc2 = acc2 + y * y
            mean_v = hsum(acc) * (1.0 / D_MODEL)
            ex2_v = hsum(acc2) * (1.0 / D_MODEL)
            var_v = ex2_v - mean_v * mean_v
            rstd = _rsqrt_nr(var_v + EPS)
            for i in range(NLG):
                rows_v[r, pl.ds(i * LANES, LANES)] = (ys[i] - mean_v) * rstd

    def fire_gather(j, s, k):
        off = s * SEQ + j * CHUNK
        pltpu.async_copy(table_hbm.at[idx_all.at[pl.ds(off, CHUNK)]],
                         rows[k], gsems[k])

    def wait_gather(k):
        pltpu.make_async_copy(table_hbm.at[idx_all.at[pl.ds(0, CHUNK)]],
                              rows[k], gsems[k]).wait()

    def fire_store(j, s, k):
        base = (wid * SEQS_PER_TILE + s) * SEQ + j * CHUNK
        pltpu.async_copy(rows[k], out_hbm.at[pl.ds(base, CHUNK)], ssems[k])

    def wait_store(k):
        pltpu.make_async_copy(rows[k], out_hbm.at[pl.ds(0, CHUNK)],
                              ssems[k]).wait()

    def chunk_body(j, _):
        pltpu.sync_copy(pos_hbm.at[pl.ds(j * CHUNK, CHUNK)], pos_v)
        for k in range(4):
            @pl.when(j > 0)
            def _(k=k):
                wait_store(k)
            fire_gather(j, k, k)

        def ring_body(u, _):
            s0 = 4 * u
            wait_gather(0)
            compute(rows0)
            fire_store(j, s0, 0)
            @pl.when(u > 0)
            def _():
                wait_store(3)
                fire_gather(j, s0 + 3, 3)
            wait_gather(1)
            compute(rows1)
            fire_store(j, s0 + 1, 1)
            wait_gather(2)
            compute(rows2)
            fire_store(j, s0 + 2, 2)
            wait_gather(3)
            compute(rows3)
            fire_store(j, s0 + 3, 3)
            @pl.when(u < SEQS_PER_TILE // 4 - 1)
            def _():
                for k in range(3):
                    wait_store(k)
                    fire_gather(j, s0 + 4 + k, k)
            return 0

        lax.fori_loop(0, SEQS_PER_TILE // 4, ring_body, 0)
        return 0

    lax.fori_loop(0, NCHUNKS, chunk_body, 0)
    for k in range(4):
        wait_store(k)


def _mask_body(labT_ref, out_ref):
    q0 = pl.program_id(0) * QB
    B = labT_ref.shape[1]
    lab3 = jnp.broadcast_to(labT_ref[...][None, :, :], (QB, SEQ, B))
    qi = lax.broadcasted_iota(jnp.int32, (QB, SEQ, B), 0) + q0
    ki = lax.broadcasted_iota(jnp.int32, (QB, SEQ, B), 1)
    out_ref[...] = jnp.logical_or(lab3 == 0, ki > qi).astype(jnp.int8)


def kernel(input_label, world_table, pos_table):
    B, S = input_label.shape
    labels_flat = input_label.reshape(-1)

    mesh = plsc.VectorSubcoreMesh(core_axis_name="c", subcore_axis_name="s")
    sc_fn = pl.kernel(
        _sc_embed_ln,
        out_type=jax.ShapeDtypeStruct((B * S, D_MODEL), jnp.float32),
        mesh=mesh,
        compiler_params=pltpu.CompilerParams(needs_layout_passes=False),
        scratch_types=(
            [pltpu.VMEM((SEQS_PER_TILE * SEQ,), jnp.int32)]
            + [pltpu.VMEM((CHUNK, D_MODEL), jnp.float32)] * 5
            + [pltpu.SemaphoreType.DMA] * 8
        ),
    )
    emb = sc_fn(labels_flat, world_table, pos_table)

    maskT = pl.pallas_call(
        _mask_body,
        grid=(S // QB,),
        in_specs=[pl.BlockSpec((S, B), lambda i: (0, 0))],
        out_specs=pl.BlockSpec((QB, S, B), lambda i: (i, 0, 0)),
        out_shape=jax.ShapeDtypeStruct((S, S, B), jnp.int8),
    )(input_label.T)

    mask = maskT.transpose(2, 0, 1).astype(jnp.bool_)
    return emb.reshape(B, S, D_MODEL), mask


# speedup vs baseline: 1.1366x; 1.1366x over previous
import jax
import jax.numpy as jnp
from jax import lax
from jax.experimental import pallas as pl
from jax.experimental.pallas import tpu as pltpu
from jax.experimental.pallas import tpu_sc as plsc

D_MODEL = 512
SEQ = 200
LANES = 16
NLG = D_MODEL // LANES
CHUNK = 40
NCHUNKS = SEQ // CHUNK
SEQS_PER_TILE = 32
EPS = 1e-5
QB = 25


def _rsqrt_nr(x):
    i = lax.bitcast_convert_type(x, jnp.int32)
    i = jnp.int32(0x5F3759DF) - (i >> 1)
    y = lax.bitcast_convert_type(i, jnp.float32)
    for _ in range(2):
        y = y * (1.5 - 0.5 * x * y * y)
    return y


def _sc_embed_ln(labels_hbm, table_hbm, pos_hbm, out_hbm,
                 idx_all, rows0, rows1, rows2, rows3, pos_v,
                 gsem0, gsem1, gsem2, gsem3, ssem0, ssem1, ssem2, ssem3):
    cid = lax.axis_index("c")
    sid = lax.axis_index("s")
    wid = sid * 2 + cid
    rows = (rows0, rows1, rows2, rows3)
    gsems = (gsem0, gsem1, gsem2, gsem3)
    ssems = (ssem0, ssem1, ssem2, ssem3)
    tile_tokens = SEQS_PER_TILE * SEQ
    pltpu.sync_copy(labels_hbm.at[pl.ds(wid * tile_tokens, tile_tokens)],
                    idx_all)

    lane = lax.iota(jnp.int32, LANES)
    perms = [(lane ^ s)[:, None] for s in (8, 4, 2, 1)]
    _gd = lax.GatherDimensionNumbers(
        offset_dims=(), collapsed_slice_dims=(0,), start_index_map=(0,))

    def hsum(v):
        for p in perms:
            v = v + lax.gather(v, p, _gd, slice_sizes=(1,),
                               mode=lax.GatherScatterMode.PROMISE_IN_BOUNDS)
        return v

    def compute(rows_v):
        @plsc.parallel_loop(0, CHUNK)
        def row_body(r):
            ys = []
            acc = jnp.zeros((LANES,), jnp.float32)
            acc2 = jnp.zeros((LANES,), jnp.float32)
            for i in range(NLG):
                x = rows_v[r, pl.ds(i * LANES, LANES)]
                p = pos_v[r, pl.ds(i * LANES, LANES)]
                y = x + p
                ys.append(y)
                acc = acc + y
                acc2 = acc2 + y * y
            mean_v = hsum(acc) * (1.0 / D_MODEL)
            ex2_v = hsum(acc2) * (1.0 / D_MODEL)
            var_v = ex2_v - mean_v * mean_v
            rstd = _rsqrt_nr(var_v + EPS)
            for i in range(NLG):
                rows_v[r, pl.ds(i * LANES, LANES)] = (ys[i] - mean_v) * rstd

    def fire_gather(j, s, k):
        off = s * SEQ + j * CHUNK
        pltpu.async_copy(table_hbm.at[idx_all.at[pl.ds(off, CHUNK)]],
                         rows[k], gsems[k])

    def wait_gather(k):
        pltpu.make_async_copy(table_hbm.at[idx_all.at[pl.ds(0, CHUNK)]],
                              rows[k], gsems[k]).wait()

    def fire_store(j, s, k):
        base = (wid * SEQS_PER_TILE + s) * SEQ + j * CHUNK
        pltpu.async_copy(rows[k], out_hbm.at[pl.ds(base, CHUNK)], ssems[k])

    def wait_store(k):
        pltpu.make_async_copy(rows[k], out_hbm.at[pl.ds(0, CHUNK)],
                              ssems[k]).wait()

    def chunk_body(j, _):
        pltpu.sync_copy(pos_hbm.at[pl.ds(j * CHUNK, CHUNK)], pos_v)
        for k in range(4):
            @pl.when(j > 0)
            def _(k=k):
                wait_store(k)
            fire_gather(j, k, k)

        def ring_body(u, _):
            s0 = 4 * u
            wait_gather(0)
            compute(rows0)
            fire_store(j, s0, 0)
            @pl.when(u > 0)
            def _():
                wait_store(3)
                fire_gather(j, s0 + 3, 3)
            wait_gather(1)
            compute(rows1)
            fire_store(j, s0 + 1, 1)
            wait_gather(2)
            compute(rows2)
            fire_store(j, s0 + 2, 2)
            wait_gather(3)
            compute(rows3)
            fire_store(j, s0 + 3, 3)
            @pl.when(u < SEQS_PER_TILE // 4 - 1)
            def _():
                for k in range(3):
                    wait_store(k)
                    fire_gather(j, s0 + 4 + k, k)
            return 0

        lax.fori_loop(0, SEQS_PER_TILE // 4, ring_body, 0)
        return 0

    lax.fori_loop(0, NCHUNKS, chunk_body, 0)
    for k in range(4):
        wait_store(k)


def _mask_body(labT_ref, out_ref):
    q0 = pl.program_id(0) * QB
    B = labT_ref.shape[1]
    lab3 = jnp.broadcast_to(labT_ref[...][None, :, :], (QB, SEQ, B))
    qi = lax.broadcasted_iota(jnp.int32, (QB, SEQ, B), 0) + q0
    ki = lax.broadcasted_iota(jnp.int32, (QB, SEQ, B), 1)
    out_ref[...] = jnp.logical_or(lab3 == 0, ki > qi).astype(jnp.int8)


def kernel(input_label, world_table, pos_table):
    B, S = input_label.shape
    labels_flat = input_label.reshape(-1)

    mesh = plsc.VectorSubcoreMesh(core_axis_name="c", subcore_axis_name="s")
    sc_fn = pl.kernel(
        _sc_embed_ln,
        out_type=jax.ShapeDtypeStruct((B * S, D_MODEL), jnp.float32),
        mesh=mesh,
        compiler_params=pltpu.CompilerParams(needs_layout_passes=False),
        scratch_types=(
            [pltpu.VMEM((SEQS_PER_TILE * SEQ,), jnp.int32)]
            + [pltpu.VMEM((CHUNK, D_MODEL), jnp.float32)] * 5
            + [pltpu.SemaphoreType.DMA] * 8
        ),
    )
    emb = sc_fn(labels_flat, world_table, pos_table)

    maskT = pl.pallas_call(
        _mask_body,
        grid=(S // QB,),
        in_specs=[pl.BlockSpec((S, B), lambda i: (0, 0))],
        out_specs=pl.BlockSpec((QB, S, B), lambda i: (i, 0, 0)),
        out_shape=jax.ShapeDtypeStruct((S, S, B), jnp.int8),
    )(input_label.T)

    mask = maskT.transpose(2, 0, 1).astype(jnp.bool_)
    return emb.reshape(B, S, D_MODEL), mask
